# baseline (speedup 1.0000x reference)
import functools

import jax
import jax.numpy as jnp
from jax import lax
from jax.experimental import pallas as pl
from jax.experimental.pallas import tpu as pltpu
from jax.experimental.pallas import tpu_sc as plsc

_N = 10000
_NP = 10240
_E = 320000
_NC = 2
_NS = 16
_NW = _NC * _NS
_B = 128
_EPW = 10240
_EP = _EPW * _NW
_CH = _EPW // _B
_CHP = _CH + 2
_RPT = _NP // _NS


def _mesh():
    return plsc.VectorSubcoreMesh(
        core_axis_name="c", subcore_axis_name="s",
        num_cores=_NC, num_subcores=_NS)


@functools.lru_cache(maxsize=None)
def _build_deg_sc():
    @functools.partial(
        pl.kernel,
        out_type=jax.ShapeDtypeStruct((_NC, 2, _NP, 16), jnp.float32),
        mesh=_mesh(),
        compiler_params=pltpu.CompilerParams(use_tc_tiling_on_sc=False),
        scratch_types=[
            pltpu.VMEM((_CHP, _B), jnp.int32),
            pltpu.VMEM((_CHP, _B), jnp.int32),
            pltpu.VMEM((_B, 16), jnp.float32),
            pltpu.VMEM_SHARED((_NP, 16), jnp.float32),
            pltpu.VMEM_SHARED((_NP, 16), jnp.float32),
            pltpu.SemaphoreType.DMA,
            pltpu.SemaphoreType.DMA,
        ],
    )
    def _deg(src_hbm, dst_hbm, ones_hbm, z_hbm, out_hbm,
             src_v, dst_v, ones_v, dsrc_sh, ddst_sh, sem0, sem1):
        cid = lax.axis_index("c")
        sid = lax.axis_index("s")
        wid = cid * _NS + sid
        pltpu.sync_copy(z_hbm.at[pl.ds(sid * _RPT, _RPT)],
                        dsrc_sh.at[pl.ds(sid * _RPT, _RPT)])
        pltpu.sync_copy(z_hbm.at[pl.ds(sid * _RPT, _RPT)],
                        ddst_sh.at[pl.ds(sid * _RPT, _RPT)])
        pltpu.sync_copy(src_hbm.at[pl.ds(wid * _CHP, _CHP)], src_v)
        pltpu.sync_copy(dst_hbm.at[pl.ds(wid * _CHP, _CHP)], dst_v)
        pltpu.sync_copy(ones_hbm, ones_v)
        plsc.subcore_barrier()

        def body(j, carry):
            pltpu.async_copy(ones_v, dsrc_sh.at[src_v.at[j]], sem0,
                             add=True)
            pltpu.async_copy(ones_v, ddst_sh.at[dst_v.at[j]], sem1,
                             add=True)
            return carry

        lax.fori_loop(0, _CH, body, 0)

        def drain(j, carry):
            pltpu.make_async_copy(ones_v, dsrc_sh.at[src_v.at[j]],
                                  sem0).wait()
            pltpu.make_async_copy(ones_v, ddst_sh.at[dst_v.at[j]],
                                  sem1).wait()
            return carry

        lax.fori_loop(0, _CH, drain, 0)
        plsc.subcore_barrier()
        pltpu.sync_copy(dsrc_sh.at[pl.ds(sid * _RPT, _RPT)],
                        out_hbm.at[cid, 0, pl.ds(sid * _RPT, _RPT)])
        pltpu.sync_copy(ddst_sh.at[pl.ds(sid * _RPT, _RPT)],
                        out_hbm.at[cid, 1, pl.ds(sid * _RPT, _RPT)])

    return _deg


def _deg_sc(src2d, dst2d, ones_rows, z16):
    return _build_deg_sc()(src2d, dst2d, ones_rows, z16)


def _conv_pass_impl(table_hbm, out_hbm, src_v, dst_v, rows, gsem, ssem,
                    table_sh, agg_sh, z_hbm, cid, sid):
    pltpu.sync_copy(table_hbm.at[pl.ds(sid * _RPT, _RPT)],
                    table_sh.at[pl.ds(sid * _RPT, _RPT)])
    pltpu.sync_copy(z_hbm.at[pl.ds(sid * _RPT, _RPT)],
                    agg_sh.at[pl.ds(sid * _RPT, _RPT)])
    plsc.subcore_barrier()

    def _gather(k, t):
        pltpu.async_copy(table_sh.at[src_v.at[k]], rows[t], gsem[t])

    def _gather_wait(k, t):
        pltpu.make_async_copy(table_sh.at[src_v.at[k]],
                              rows[t], gsem[t]).wait()

    def _scatter(k, t):
        pltpu.async_copy(rows[t], agg_sh.at[dst_v.at[k]], ssem[t],
                         add=True)

    def _scatter_wait(k, t):
        pltpu.make_async_copy(rows[t], agg_sh.at[dst_v.at[k]],
                              ssem[t]).wait()

    for t in range(3):
        _gather(t, t)
    for t in range(3):
        _gather_wait(t, t)
        _scatter(t, t)

    def body(i, carry):
        for t in range(3):
            k = 3 * i + 3 + t
            _scatter_wait(k - 3, t)
            _gather(k, t)
        for t in range(3):
            k = 3 * i + 3 + t
            _gather_wait(k, t)
            _scatter(k, t)
        return carry

    lax.fori_loop(0, (_CH - 2) // 3, body, 0)
    for t in range(3):
        _scatter_wait(_CH - 2 + t, t)
    plsc.subcore_barrier()
    pltpu.sync_copy(agg_sh.at[pl.ds(sid * _RPT, _RPT)],
                    out_hbm.at[cid, pl.ds(sid * _RPT, _RPT)])


_CONV_SCRATCH = [
    pltpu.VMEM((_CHP, _B), jnp.int32),
    pltpu.VMEM((_CHP, _B), jnp.int32),
    pltpu.VMEM((_B, 64), jnp.float32),
    pltpu.VMEM((_B, 64), jnp.float32),
    pltpu.VMEM((_B, 64), jnp.float32),
    pltpu.VMEM_SHARED((_NP, 64), jnp.float32),
    pltpu.VMEM_SHARED((_NP, 64), jnp.float32),
    pltpu.SemaphoreType.DMA,
    pltpu.SemaphoreType.DMA,
    pltpu.SemaphoreType.DMA,
    pltpu.SemaphoreType.DMA,
    pltpu.SemaphoreType.DMA,
    pltpu.SemaphoreType.DMA,
]


@functools.lru_cache(maxsize=None)
def _build_conv_sc():
    @functools.partial(
        pl.kernel,
        out_type=jax.ShapeDtypeStruct((_NC, _NP, 64), jnp.float32),
        mesh=_mesh(),
        compiler_params=pltpu.CompilerParams(use_tc_tiling_on_sc=False),
        scratch_types=_CONV_SCRATCH,
    )
    def _conv(table_hbm, src_hbm, dst_hbm, z_hbm, out_hbm,
              src_v, dst_v, r0, r1, r2, table_sh, agg_sh,
              g0, g1, g2, s0, s1, s2):
        cid = lax.axis_index("c")
        sid = lax.axis_index("s")
        wid = cid * _NS + sid
        pltpu.sync_copy(src_hbm.at[pl.ds(wid * _CHP, _CHP)], src_v)
        pltpu.sync_copy(dst_hbm.at[pl.ds(wid * _CHP, _CHP)], dst_v)
        _conv_pass_impl(table_hbm, out_hbm, src_v, dst_v, (r0, r1, r2),
                        (g0, g1, g2), (s0, s1, s2), table_sh, agg_sh,
                        z_hbm, cid, sid)

    return _conv


@functools.lru_cache(maxsize=None)
def _build_conv2_sc():
    @functools.partial(
        pl.kernel,
        out_type=(jax.ShapeDtypeStruct((_NC, _NP, 64), jnp.float32),
                  jax.ShapeDtypeStruct((_NC, _NP, 64), jnp.float32)),
        mesh=_mesh(),
        compiler_params=pltpu.CompilerParams(use_tc_tiling_on_sc=False),
        scratch_types=_CONV_SCRATCH,
    )
    def _conv2(ta_hbm, tb_hbm, src_hbm, dst_hbm, z_hbm, outa_hbm, outb_hbm,
               src_v, dst_v, r0, r1, r2, table_sh, agg_sh,
               g0, g1, g2, s0, s1, s2):
        cid = lax.axis_index("c")
        sid = lax.axis_index("s")
        wid = cid * _NS + sid
        pltpu.sync_copy(src_hbm.at[pl.ds(wid * _CHP, _CHP)], src_v)
        pltpu.sync_copy(dst_hbm.at[pl.ds(wid * _CHP, _CHP)], dst_v)
        _conv_pass_impl(ta_hbm, outa_hbm, src_v, dst_v, (r0, r1, r2),
                        (g0, g1, g2), (s0, s1, s2), table_sh, agg_sh,
                        z_hbm, cid, sid)
        plsc.subcore_barrier()
        _conv_pass_impl(tb_hbm, outb_hbm, src_v, dst_v, (r0, r1, r2),
                        (g0, g1, g2), (s0, s1, s2), table_sh, agg_sh,
                        z_hbm, cid, sid)

    return _conv2


def _conv_sc64(table, src2d, dst2d, z):
    return _build_conv_sc()(table, src2d, dst2d, z)


def _conv2_sc64(ta, tb, src2d, dst2d, z):
    return _build_conv2_sc()(ta, tb, src2d, dst2d, z)


def _lrelu(x):
    return jnp.where(x > 0, x, 0.01 * x)


def _bn(x, g, b, eps=1e-5):
    m = jnp.mean(x, axis=0, keepdims=True)
    v = jnp.mean((x - m) * (x - m), axis=0, keepdims=True)
    return g * (x - m) * lax.rsqrt(v + eps) + b


def _store_padded(ref, val):
    ref[: _N] = val
    ref[_N:] = jnp.zeros((_NP - _N, val.shape[1]), jnp.float32)


def _tc_a0(nf, w1, u_ref):
    u_ref[...] = jnp.dot(nf[...], w1[...],
                         preferred_element_type=jnp.float32)


def _tc_a(u, degs, t1_ref, nsrc_ref, ndst_ref):
    ds = degs[0, 0, : _N, 0:1] + degs[1, 0, : _N, 0:1]
    dd = degs[0, 1, : _N, 0:1] + degs[1, 1, : _N, 0:1]
    ns = lax.rsqrt(jnp.maximum(ds, 1.0))
    nd = lax.rsqrt(jnp.maximum(dd, 1.0))
    nsrc_ref[...] = ns
    ndst_ref[...] = nd
    _store_padded(t1_ref, u[...] * ns)


def _tc_b(a, nd, b1, g1, be1, w2, ns, t2_ref):
    x = (a[0, : _N] + a[1, : _N]) * nd[...] + b1[...]
    h = _lrelu(_bn(x, g1[...], be1[...]))
    _store_padded(t2_ref, jnp.dot(h, w2[...],
                                  preferred_element_type=jnp.float32) * ns[...])


def _tc_c(a, nd, b2, g2, be2, wn1, wg1a, wg1b, nt, ns, t3_ref, tg_ref):
    x = (a[0, : _N] + a[1, : _N]) * nd[...] + b2[...]
    h = _lrelu(_bn(x, g2[...], be2[...]))
    _store_padded(t3_ref, jnp.dot(h, wn1[...],
                                  preferred_element_type=jnp.float32) * ns[...])
    _store_padded(tg_ref,
                  (jnp.dot(h, wg1a[...], preferred_element_type=jnp.float32)
                   + nt[...] * wg1b[...]) * ns[...])


def _tc_d(a, g, nd, bn1, bg1, wn2, ns, t4_ref, gmean_ref):
    hn = _lrelu((a[0, : _N] + a[1, : _N]) * nd[...] + bn1[...])
    _store_padded(t4_ref, jnp.dot(hn, wn2[...],
                                  preferred_element_type=jnp.float32) * ns[...])
    hg = _lrelu((g[0, : _N] + g[1, : _N]) * nd[...] + bg1[...])
    gmean_ref[...] = jnp.mean(hg, axis=0, keepdims=True)


def _tc_e(a, nd, bn2, wnc, bnc, gmean, wgc, bgc, nout_ref, gout_ref):
    hn = _lrelu((a[0, : _N] + a[1, : _N]) * nd[...] + bn2[...])
    nout_ref[...] = jnp.dot(hn, wnc[...],
                            preferred_element_type=jnp.float32) + bnc[...]
    gout_ref[...] = jnp.dot(gmean[...], wgc[...],
                            preferred_element_type=jnp.float32) + bgc[...]


def _call(fn, out_shapes, *args):
    return pl.pallas_call(
        fn, out_shape=[jax.ShapeDtypeStruct(s, jnp.float32)
                       for s in out_shapes])(*args)


def _pad_rows(t):
    return jnp.concatenate(
        [t, jnp.zeros((_NP - _N, t.shape[1]), jnp.float32)], axis=0)


def kernel(node_feat, nodetype, edge_feat, edge_index,
           W1, b1, g1, be1, W2, b2, g2, be2, We, bE,
           Wn1, bn1, Wn2, bn2, Wnc, bnc, Wg1, bg1, Wgc, bgc):
    f32 = jnp.float32
    src = edge_index[0]
    dst = edge_index[1]
    pad = jnp.full((_EP - _E,), _N, jnp.int32)
    extra = jnp.full((_NW, (_CHP - _CH) * _B), _N, jnp.int32)

    def _chunked(idx):
        w = jnp.concatenate([idx, pad]).reshape(_NW, _CH * _B)
        return jnp.concatenate([w, extra], axis=1).reshape(_NW * _CHP, _B)

    src2d = _chunked(src)
    dst2d = _chunked(dst)
    z16 = jnp.zeros((_NP, 16), f32)
    z64 = jnp.zeros((_NP, 64), f32)
    ones_rows = jnp.ones((_B, 16), f32)

    degs = _deg_sc(src2d, dst2d, ones_rows, z16)
    (u1,) = _call(_tc_a0, [(_N, 64)], node_feat, W1)
    t1, nsrc, ndst = _call(_tc_a, [(_NP, 64), (_N, 1), (_N, 1)], u1, degs)

    a = _conv_sc64(t1, src2d, dst2d, z64)
    (t2,) = _call(_tc_b, [(_NP, 64)],
                  a, ndst, b1[None, :], g1[None, :],
                  be1[None, :], W2, nsrc)

    a = _conv_sc64(t2, src2d, dst2d, z64)
    t3, tg = _call(_tc_c, [(_NP, 64), (_NP, 64)],
                   a, ndst, b2[None, :], g2[None, :],
                   be2[None, :], Wn1, Wg1[:64], Wg1[64:65],
                   nodetype[:, None], nsrc)

    a, ag = _conv2_sc64(t3, tg, src2d, dst2d, z64)
    t4, gmean = _call(_tc_d, [(_NP, 64), (1, 64)],
                      a, ag, ndst,
                      bn1[None, :], bg1[None, :], Wn2, nsrc)

    a = _conv_sc64(t4, src2d, dst2d, z64)
    node_out, graph_out = _call(_tc_e, [(_N, 16), (1, 8)],
                                a, ndst, bn2[None, :],
                                Wnc, bnc[None, :], gmean, Wgc, bgc[None, :])
    return node_out, graph_out


# speedup vs baseline: 1.1705x; 1.1705x over previous
import functools

import jax
import jax.numpy as jnp
from jax import lax
from jax.experimental import pallas as pl
from jax.experimental.pallas import tpu as pltpu
from jax.experimental.pallas import tpu_sc as plsc

_N = 10000
_NP = 10240
_E = 320000
_NC = 2
_NS = 16
_NW = _NC * _NS
_B = 128
_EPW = 10240
_EP = _EPW * _NW
_CH = _EPW // _B
_CHP = _CH + 2
_RPT = _NP // _NS


def _mesh():
    return plsc.VectorSubcoreMesh(
        core_axis_name="c", subcore_axis_name="s",
        num_cores=_NC, num_subcores=_NS)


@functools.lru_cache(maxsize=None)
def _build_deg_sc():
    @functools.partial(
        pl.kernel,
        out_type=jax.ShapeDtypeStruct((_NC, 2, _NP, 16), jnp.float32),
        mesh=_mesh(),
        compiler_params=pltpu.CompilerParams(use_tc_tiling_on_sc=False),
        scratch_types=[
            pltpu.VMEM((_CHP, _B), jnp.int32),
            pltpu.VMEM((_CHP, _B), jnp.int32),
            pltpu.VMEM((_B, 16), jnp.float32),
            pltpu.VMEM_SHARED((_NP, 16), jnp.float32),
            pltpu.VMEM_SHARED((_NP, 16), jnp.float32),
            pltpu.SemaphoreType.DMA,
            pltpu.SemaphoreType.DMA,
        ],
    )
    def _deg(src_hbm, dst_hbm, ones_hbm, z_hbm, out_hbm,
             src_v, dst_v, ones_v, dsrc_sh, ddst_sh, sem0, sem1):
        cid = lax.axis_index("c")
        sid = lax.axis_index("s")
        wid = cid * _NS + sid
        pltpu.sync_copy(z_hbm.at[pl.ds(sid * _RPT, _RPT)],
                        dsrc_sh.at[pl.ds(sid * _RPT, _RPT)])
        pltpu.sync_copy(z_hbm.at[pl.ds(sid * _RPT, _RPT)],
                        ddst_sh.at[pl.ds(sid * _RPT, _RPT)])
        pltpu.sync_copy(src_hbm.at[pl.ds(wid * _CHP, _CHP)], src_v)
        pltpu.sync_copy(dst_hbm.at[pl.ds(wid * _CHP, _CHP)], dst_v)
        pltpu.sync_copy(ones_hbm, ones_v)
        plsc.subcore_barrier()

        def body(j, carry):
            pltpu.async_copy(ones_v, dsrc_sh.at[src_v.at[j]], sem0,
                             add=True)
            pltpu.async_copy(ones_v, ddst_sh.at[dst_v.at[j]], sem1,
                             add=True)
            return carry

        lax.fori_loop(0, _CH, body, 0)

        def drain(j, carry):
            pltpu.make_async_copy(ones_v, dsrc_sh.at[src_v.at[j]],
                                  sem0).wait()
            pltpu.make_async_copy(ones_v, ddst_sh.at[dst_v.at[j]],
                                  sem1).wait()
            return carry

        lax.fori_loop(0, _CH, drain, 0)
        plsc.subcore_barrier()
        pltpu.sync_copy(dsrc_sh.at[pl.ds(sid * _RPT, _RPT)],
                        out_hbm.at[cid, 0, pl.ds(sid * _RPT, _RPT)])
        pltpu.sync_copy(ddst_sh.at[pl.ds(sid * _RPT, _RPT)],
                        out_hbm.at[cid, 1, pl.ds(sid * _RPT, _RPT)])

    return _deg


def _deg_sc(src2d, dst2d, ones_rows, z16):
    return _build_deg_sc()(src2d, dst2d, ones_rows, z16)


def _conv_pass_impl(table_hbm, out_hbm, src_v, dst_v, rows, gsem, ssem,
                    table_sh, agg_sh, z_hbm, cid, sid):
    pltpu.sync_copy(table_hbm.at[pl.ds(sid * _RPT, _RPT)],
                    table_sh.at[pl.ds(sid * _RPT, _RPT)])
    pltpu.sync_copy(z_hbm.at[pl.ds(sid * _RPT, _RPT)],
                    agg_sh.at[pl.ds(sid * _RPT, _RPT)])
    plsc.subcore_barrier()

    rows0, rows1 = rows[0], rows[1]
    gsem0, gsem1 = gsem[0], gsem[1]
    pltpu.async_copy(table_sh.at[src_v.at[0]], rows0, gsem0)

    def body(i, carry):
        j0 = 2 * i
        pltpu.async_copy(table_sh.at[src_v.at[j0 + 1]], rows1, gsem1)
        pltpu.make_async_copy(table_sh.at[src_v.at[j0]],
                              rows0, gsem0).wait()
        pltpu.sync_copy(rows0, agg_sh.at[dst_v.at[j0]], add=True)
        pltpu.async_copy(table_sh.at[src_v.at[j0 + 2]], rows0, gsem0)
        pltpu.make_async_copy(table_sh.at[src_v.at[j0 + 1]],
                              rows1, gsem1).wait()
        pltpu.sync_copy(rows1, agg_sh.at[dst_v.at[j0 + 1]], add=True)
        return carry

    lax.fori_loop(0, _CH // 2, body, 0)
    pltpu.make_async_copy(table_sh.at[src_v.at[_CH]],
                          rows0, gsem0).wait()
    plsc.subcore_barrier()
    pltpu.sync_copy(agg_sh.at[pl.ds(sid * _RPT, _RPT)],
                    out_hbm.at[cid, pl.ds(sid * _RPT, _RPT)])


_CONV_SCRATCH = [
    pltpu.VMEM((_CHP, _B), jnp.int32),
    pltpu.VMEM((_CHP, _B), jnp.int32),
    pltpu.VMEM((_B, 64), jnp.float32),
    pltpu.VMEM((_B, 64), jnp.float32),
    pltpu.VMEM_SHARED((_NP, 64), jnp.float32),
    pltpu.VMEM_SHARED((_NP, 64), jnp.float32),
    pltpu.SemaphoreType.DMA,
    pltpu.SemaphoreType.DMA,
]


@functools.lru_cache(maxsize=None)
def _build_conv_sc():
    @functools.partial(
        pl.kernel,
        out_type=jax.ShapeDtypeStruct((_NC, _NP, 64), jnp.float32),
        mesh=_mesh(),
        compiler_params=pltpu.CompilerParams(use_tc_tiling_on_sc=False),
        scratch_types=_CONV_SCRATCH,
    )
    def _conv(table_hbm, src_hbm, dst_hbm, z_hbm, out_hbm,
              src_v, dst_v, r0, r1, table_sh, agg_sh, g0, g1):
        cid = lax.axis_index("c")
        sid = lax.axis_index("s")
        wid = cid * _NS + sid
        pltpu.sync_copy(src_hbm.at[pl.ds(wid * _CHP, _CHP)], src_v)
        pltpu.sync_copy(dst_hbm.at[pl.ds(wid * _CHP, _CHP)], dst_v)
        _conv_pass_impl(table_hbm, out_hbm, src_v, dst_v, (r0, r1),
                        (g0, g1), None, table_sh, agg_sh, z_hbm, cid, sid)

    return _conv


@functools.lru_cache(maxsize=None)
def _build_conv2_sc():
    @functools.partial(
        pl.kernel,
        out_type=(jax.ShapeDtypeStruct((_NC, _NP, 64), jnp.float32),
                  jax.ShapeDtypeStruct((_NC, _NP, 64), jnp.float32)),
        mesh=_mesh(),
        compiler_params=pltpu.CompilerParams(use_tc_tiling_on_sc=False),
        scratch_types=_CONV_SCRATCH,
    )
    def _conv2(ta_hbm, tb_hbm, src_hbm, dst_hbm, z_hbm, outa_hbm, outb_hbm,
               src_v, dst_v, r0, r1, table_sh, agg_sh, g0, g1):
        cid = lax.axis_index("c")
        sid = lax.axis_index("s")
        wid = cid * _NS + sid
        pltpu.sync_copy(src_hbm.at[pl.ds(wid * _CHP, _CHP)], src_v)
        pltpu.sync_copy(dst_hbm.at[pl.ds(wid * _CHP, _CHP)], dst_v)
        _conv_pass_impl(ta_hbm, outa_hbm, src_v, dst_v, (r0, r1),
                        (g0, g1), None, table_sh, agg_sh, z_hbm, cid, sid)
        plsc.subcore_barrier()
        _conv_pass_impl(tb_hbm, outb_hbm, src_v, dst_v, (r0, r1),
                        (g0, g1), None, table_sh, agg_sh, z_hbm, cid, sid)

    return _conv2


def _conv_sc64(table, src2d, dst2d, z):
    return _build_conv_sc()(table, src2d, dst2d, z)


def _conv2_sc64(ta, tb, src2d, dst2d, z):
    return _build_conv2_sc()(ta, tb, src2d, dst2d, z)


def _lrelu(x):
    return jnp.where(x > 0, x, 0.01 * x)


def _bn(x, g, b, eps=1e-5):
    m = jnp.mean(x, axis=0, keepdims=True)
    v = jnp.mean((x - m) * (x - m), axis=0, keepdims=True)
    return g * (x - m) * lax.rsqrt(v + eps) + b


def _store_padded(ref, val):
    ref[: _N] = val
    ref[_N:] = jnp.zeros((_NP - _N, val.shape[1]), jnp.float32)


def _tc_a0(nf, w1, u_ref):
    u_ref[...] = jnp.dot(nf[...], w1[...],
                         preferred_element_type=jnp.float32)


def _tc_a(u, degs, t1_ref, nsrc_ref, ndst_ref):
    ds = degs[0, 0, : _N, 0:1] + degs[1, 0, : _N, 0:1]
    dd = degs[0, 1, : _N, 0:1] + degs[1, 1, : _N, 0:1]
    ns = lax.rsqrt(jnp.maximum(ds, 1.0))
    nd = lax.rsqrt(jnp.maximum(dd, 1.0))
    nsrc_ref[...] = ns
    ndst_ref[...] = nd
    _store_padded(t1_ref, u[...] * ns)


def _tc_b(a, nd, b1, g1, be1, w2, ns, t2_ref):
    x = (a[0, : _N] + a[1, : _N]) * nd[...] + b1[...]
    h = _lrelu(_bn(x, g1[...], be1[...]))
    _store_padded(t2_ref, jnp.dot(h, w2[...],
                                  preferred_element_type=jnp.float32) * ns[...])


def _tc_c(a, nd, b2, g2, be2, wn1, wg1a, wg1b, nt, ns, t3_ref, tg_ref):
    x = (a[0, : _N] + a[1, : _N]) * nd[...] + b2[...]
    h = _lrelu(_bn(x, g2[...], be2[...]))
    _store_padded(t3_ref, jnp.dot(h, wn1[...],
                                  preferred_element_type=jnp.float32) * ns[...])
    _store_padded(tg_ref,
                  (jnp.dot(h, wg1a[...], preferred_element_type=jnp.float32)
                   + nt[...] * wg1b[...]) * ns[...])


def _tc_d(a, g, nd, bn1, bg1, wn2, ns, t4_ref, gmean_ref):
    hn = _lrelu((a[0, : _N] + a[1, : _N]) * nd[...] + bn1[...])
    _store_padded(t4_ref, jnp.dot(hn, wn2[...],
                                  preferred_element_type=jnp.float32) * ns[...])
    hg = _lrelu((g[0, : _N] + g[1, : _N]) * nd[...] + bg1[...])
    gmean_ref[...] = jnp.mean(hg, axis=0, keepdims=True)


def _tc_e(a, nd, bn2, wnc, bnc, gmean, wgc, bgc, nout_ref, gout_ref):
    hn = _lrelu((a[0, : _N] + a[1, : _N]) * nd[...] + bn2[...])
    nout_ref[...] = jnp.dot(hn, wnc[...],
                            preferred_element_type=jnp.float32) + bnc[...]
    gout_ref[...] = jnp.dot(gmean[...], wgc[...],
                            preferred_element_type=jnp.float32) + bgc[...]


def _call(fn, out_shapes, *args):
    return pl.pallas_call(
        fn, out_shape=[jax.ShapeDtypeStruct(s, jnp.float32)
                       for s in out_shapes])(*args)


def _pad_rows(t):
    return jnp.concatenate(
        [t, jnp.zeros((_NP - _N, t.shape[1]), jnp.float32)], axis=0)


def kernel(node_feat, nodetype, edge_feat, edge_index,
           W1, b1, g1, be1, W2, b2, g2, be2, We, bE,
           Wn1, bn1, Wn2, bn2, Wnc, bnc, Wg1, bg1, Wgc, bgc):
    f32 = jnp.float32
    src = edge_index[0]
    dst = edge_index[1]
    pad = jnp.full((_EP - _E,), _N, jnp.int32)
    extra = jnp.full((_NW, (_CHP - _CH) * _B), _N, jnp.int32)

    def _chunked(idx):
        w = jnp.concatenate([idx, pad]).reshape(_NW, _CH * _B)
        return jnp.concatenate([w, extra], axis=1).reshape(_NW * _CHP, _B)

    src2d = _chunked(src)
    dst2d = _chunked(dst)
    z16 = jnp.zeros((_NP, 16), f32)
    z64 = jnp.zeros((_NP, 64), f32)
    ones_rows = jnp.ones((_B, 16), f32)

    degs = _deg_sc(src2d, dst2d, ones_rows, z16)
    (u1,) = _call(_tc_a0, [(_N, 64)], node_feat, W1)
    t1, nsrc, ndst = _call(_tc_a, [(_NP, 64), (_N, 1), (_N, 1)], u1, degs)

    a = _conv_sc64(t1, src2d, dst2d, z64)
    (t2,) = _call(_tc_b, [(_NP, 64)],
                  a, ndst, b1[None, :], g1[None, :],
                  be1[None, :], W2, nsrc)

    a = _conv_sc64(t2, src2d, dst2d, z64)
    t3, tg = _call(_tc_c, [(_NP, 64), (_NP, 64)],
                   a, ndst, b2[None, :], g2[None, :],
                   be2[None, :], Wn1, Wg1[:64], Wg1[64:65],
                   nodetype[:, None], nsrc)

    a, ag = _conv2_sc64(t3, tg, src2d, dst2d, z64)
    t4, gmean = _call(_tc_d, [(_NP, 64), (1, 64)],
                      a, ag, ndst,
                      bn1[None, :], bg1[None, :], Wn2, nsrc)

    a = _conv_sc64(t4, src2d, dst2d, z64)
    node_out, graph_out = _call(_tc_e, [(_N, 16), (1, 8)],
                                a, ndst, bn2[None, :],
                                Wnc, bnc[None, :], gmean, Wgc, bgc[None, :])
    return node_out, graph_out
